# baseline (speedup 1.0000x reference)
import jax
import jax.numpy as jnp
from jax.experimental import pallas as pl
from jax.experimental.pallas import tpu as pltpu

_DIM = 2048
_NUM_EXPERTS = 64
_K = 2


def _router_kernel(x_ref, whi_ref, wlo_ref, b_ref, scores_ref, idx_ref, vals_ref):
    x = x_ref[...]
    xhi = x.astype(jnp.bfloat16)
    xlo = (x - xhi.astype(jnp.float32)).astype(jnp.bfloat16)
    whi = whi_ref[...]
    wlo = wlo_ref[...]
    logits = jnp.dot(xhi, whi, preferred_element_type=jnp.float32)
    logits = logits + jnp.dot(xlo, whi, preferred_element_type=jnp.float32)
    logits = logits + jnp.dot(xhi, wlo, preferred_element_type=jnp.float32)
    logits = logits + b_ref[...]

    m = jnp.max(logits, axis=-1, keepdims=True)
    e = jnp.exp(logits - m)
    s = jnp.sum(e, axis=-1, keepdims=True)
    p = e / s
    scores_ref[...] = p

    blk = p.shape[0]
    iota = jax.lax.broadcasted_iota(jnp.int32, (blk, _NUM_EXPERTS), 1)

    v1 = jnp.max(p, axis=-1, keepdims=True)
    i1 = jnp.min(jnp.where(p == v1, iota, _NUM_EXPERTS), axis=-1, keepdims=True)
    masked = jnp.where(iota == i1, -jnp.inf, p)
    v2 = jnp.max(masked, axis=-1, keepdims=True)
    i2 = jnp.min(
        jnp.where(masked == v2, iota, _NUM_EXPERTS), axis=-1, keepdims=True
    )

    vals_ref[...] = jnp.concatenate([v1, v2], axis=-1)
    idx_ref[...] = jnp.concatenate([i1, i2], axis=-1)


@jax.jit
def kernel(x, W, b):
    n_tokens, dim = x.shape
    num_experts = W.shape[0]
    assert dim == _DIM and num_experts == _NUM_EXPERTS

    block = 1024
    grid = (n_tokens // block,)

    wt = W.T
    whi = wt.astype(jnp.bfloat16)
    wlo = (wt - whi.astype(jnp.float32)).astype(jnp.bfloat16)
    b2 = b.reshape(1, num_experts)

    scores, idx, vals = pl.pallas_call(
        _router_kernel,
        grid=grid,
        in_specs=[
            pl.BlockSpec((block, dim), lambda i: (i, 0)),
            pl.BlockSpec((dim, num_experts), lambda i: (0, 0)),
            pl.BlockSpec((dim, num_experts), lambda i: (0, 0)),
            pl.BlockSpec((1, num_experts), lambda i: (0, 0)),
        ],
        out_specs=[
            pl.BlockSpec((block, num_experts), lambda i: (i, 0)),
            pl.BlockSpec((block, _K), lambda i: (i, 0)),
            pl.BlockSpec((block, _K), lambda i: (i, 0)),
        ],
        out_shape=[
            jax.ShapeDtypeStruct((n_tokens, num_experts), jnp.float32),
            jax.ShapeDtypeStruct((n_tokens, _K), jnp.int32),
            jax.ShapeDtypeStruct((n_tokens, _K), jnp.float32),
        ],
        compiler_params=pltpu.CompilerParams(
            dimension_semantics=("parallel",),
        ),
    )(x, whi, wlo, b2)

    return scores, idx, vals


# speedup vs baseline: 1.1196x; 1.1196x over previous
import jax
import jax.numpy as jnp
from jax.experimental import pallas as pl
from jax.experimental.pallas import tpu as pltpu

_DIM = 2048
_NUM_EXPERTS = 64
_K = 2


def _router_kernel(x_ref, wt_ref, b_ref, scores_ref, idx_ref, vals_ref):
    x = x_ref[...]
    logits = jnp.dot(x, wt_ref[...], preferred_element_type=jnp.float32)
    logits = logits + b_ref[...]

    m = jnp.max(logits, axis=-1, keepdims=True)
    e = jnp.exp(logits - m)
    s = jnp.sum(e, axis=-1, keepdims=True)
    p = e / s
    scores_ref[...] = p

    blk = p.shape[0]
    iota = jax.lax.broadcasted_iota(jnp.int32, (blk, _NUM_EXPERTS), 1)

    v1 = jnp.max(p, axis=-1, keepdims=True)
    i1 = jnp.min(jnp.where(p == v1, iota, _NUM_EXPERTS), axis=-1, keepdims=True)
    masked = jnp.where(iota == i1, -jnp.inf, p)
    v2 = jnp.max(masked, axis=-1, keepdims=True)
    i2 = jnp.min(
        jnp.where(masked == v2, iota, _NUM_EXPERTS), axis=-1, keepdims=True
    )

    vals_ref[...] = jnp.concatenate([v1, v2], axis=-1)
    idx_ref[...] = jnp.concatenate([i1, i2], axis=-1)


@jax.jit
def kernel(x, W, b):
    n_tokens, dim = x.shape
    num_experts = W.shape[0]
    assert dim == _DIM and num_experts == _NUM_EXPERTS

    block = 512
    grid = (n_tokens // block,)

    wt = W.T
    b2 = b.reshape(1, num_experts)

    scores, idx, vals = pl.pallas_call(
        _router_kernel,
        grid=grid,
        in_specs=[
            pl.BlockSpec((block, dim), lambda i: (i, 0)),
            pl.BlockSpec((dim, num_experts), lambda i: (0, 0)),
            pl.BlockSpec((1, num_experts), lambda i: (0, 0)),
        ],
        out_specs=[
            pl.BlockSpec((block, num_experts), lambda i: (i, 0)),
            pl.BlockSpec((block, _K), lambda i: (i, 0)),
            pl.BlockSpec((block, _K), lambda i: (i, 0)),
        ],
        out_shape=[
            jax.ShapeDtypeStruct((n_tokens, num_experts), jnp.float32),
            jax.ShapeDtypeStruct((n_tokens, _K), jnp.int32),
            jax.ShapeDtypeStruct((n_tokens, _K), jnp.float32),
        ],
        compiler_params=pltpu.CompilerParams(
            dimension_semantics=("parallel",),
        ),
    )(x, wt, b2)

    return scores, idx, vals


# speedup vs baseline: 1.3334x; 1.1910x over previous
import jax
import jax.numpy as jnp
from jax.experimental import pallas as pl
from jax.experimental.pallas import tpu as pltpu

_DIM = 2048
_NUM_EXPERTS = 64
_K = 2


def _router_kernel(x_ref, wt_ref, b_ref, scores_ref, idx_ref, vals_ref):
    x = x_ref[...]
    logits = jnp.dot(x, wt_ref[...], preferred_element_type=jnp.float32)
    logits = logits + b_ref[...]

    m = jnp.max(logits, axis=-1, keepdims=True)
    e = jnp.exp(logits - m)
    s = jnp.sum(e, axis=-1, keepdims=True)
    p = e / s
    scores_ref[...] = p

    blk = p.shape[0]
    iota = jax.lax.broadcasted_iota(jnp.int32, (blk, _NUM_EXPERTS), 1)

    v1 = jnp.max(p, axis=-1, keepdims=True)
    i1 = jnp.min(jnp.where(p == v1, iota, _NUM_EXPERTS), axis=-1, keepdims=True)
    masked = jnp.where(iota == i1, -jnp.inf, p)
    v2 = jnp.max(masked, axis=-1, keepdims=True)
    i2 = jnp.min(
        jnp.where(masked == v2, iota, _NUM_EXPERTS), axis=-1, keepdims=True
    )

    vals_ref[...] = jnp.concatenate([v1, v2], axis=-1)
    idx_ref[...] = jnp.concatenate([i1, i2], axis=-1)


@jax.jit
def kernel(x, W, b):
    n_tokens, dim = x.shape
    num_experts = W.shape[0]
    assert dim == _DIM and num_experts == _NUM_EXPERTS

    block = 2048
    grid = (n_tokens // block,)

    wt = W.T
    b2 = b.reshape(1, num_experts)

    scores, idx, vals = pl.pallas_call(
        _router_kernel,
        grid=grid,
        in_specs=[
            pl.BlockSpec((block, dim), lambda i: (i, 0)),
            pl.BlockSpec((dim, num_experts), lambda i: (0, 0)),
            pl.BlockSpec((1, num_experts), lambda i: (0, 0)),
        ],
        out_specs=[
            pl.BlockSpec((block, num_experts), lambda i: (i, 0)),
            pl.BlockSpec((block, _K), lambda i: (i, 0)),
            pl.BlockSpec((block, _K), lambda i: (i, 0)),
        ],
        out_shape=[
            jax.ShapeDtypeStruct((n_tokens, num_experts), jnp.float32),
            jax.ShapeDtypeStruct((n_tokens, _K), jnp.int32),
            jax.ShapeDtypeStruct((n_tokens, _K), jnp.float32),
        ],
        compiler_params=pltpu.CompilerParams(
            dimension_semantics=("parallel",),
        ),
    )(x, wt, b2)

    return scores, idx, vals
